# Initial kernel scaffold; baseline (speedup 1.0000x reference)
import jax
import jax.numpy as jnp
from jax.experimental import pallas as pl


def kernel(x, edge_index, edge_type, W1, b1, W2, b2, W3, b3):
    raise NotImplementedError("write your pallas kernel here")


# speedup vs baseline: 6.5314x; 6.5314x over previous
import functools

import jax
import jax.numpy as jnp
from jax import lax
from jax.experimental import pallas as pl
from jax.experimental.pallas import tpu as pltpu
from jax.experimental.pallas import tpu_sc as plsc

N = 10000
E = 160000
D = 256
DH = 128
CH = 128
EP = 163840
NCHUNK = EP // CH
CPT = NCHUNK // 16
CPT_DEG = NCHUNK // 32
RPT = 632
NP = 16 * RPT
BM = 1000

_mesh = plsc.VectorSubcoreMesh(core_axis_name="c", subcore_axis_name="s")


def _deg_body(col_hbm, ones_hbm, zeros_hbm, out_hbm, col_v, ones_v, acc_sh):
    c = lax.axis_index("c")
    s = lax.axis_index("s")
    base = s * RPT
    pltpu.sync_copy(zeros_hbm.at[pl.ds(base, RPT)], acc_sh.at[pl.ds(base, RPT)])
    pltpu.sync_copy(ones_hbm, ones_v)
    wid = c * 16 + s
    pltpu.sync_copy(col_hbm.at[pl.ds(wid * CPT_DEG, CPT_DEG)], col_v)
    plsc.subcore_barrier()

    def chunk(j, carry):
        pltpu.sync_copy(ones_v, acc_sh.at[col_v.at[j]], add=True)
        return carry

    lax.fori_loop(0, CPT_DEG, chunk, 0)
    plsc.subcore_barrier()
    pltpu.sync_copy(acc_sh.at[pl.ds(base, RPT)], out_hbm.at[c, pl.ds(base, RPT)])


_sc_degree = pl.kernel(
    _deg_body,
    mesh=_mesh,
    out_type=jax.ShapeDtypeStruct((2, NP, DH), jnp.float32),
    scratch_types=[
        pltpu.VMEM((CPT_DEG, CH), jnp.int32),
        pltpu.VMEM((CH, DH), jnp.float32),
        pltpu.VMEM_SHARED((NP, DH), jnp.float32),
    ],
)


def _scat_body(table_hbm, row_hbm, col_hbm, zeros_hbm, out_hbm,
               row_v, col_v, gidx, rows_v, acc_sh, sem):
    c = lax.axis_index("c")
    s = lax.axis_index("s")
    base = s * RPT
    pltpu.sync_copy(zeros_hbm.at[pl.ds(base, RPT)], acc_sh.at[pl.ds(base, RPT)])
    pltpu.sync_copy(row_hbm.at[pl.ds(s * CPT, CPT)], row_v)
    pltpu.sync_copy(col_hbm.at[pl.ds(s * CPT, CPT)], col_v)
    plsc.subcore_barrier()
    off = c * N

    def chunk(j, carry):
        for k in range(CH // 16):
            gidx[pl.ds(k * 16, 16)] = row_v[j, pl.ds(k * 16, 16)] + off
        pltpu.async_copy(table_hbm.at[gidx], rows_v, sem).wait()
        pltpu.sync_copy(rows_v, acc_sh.at[col_v.at[j]], add=True)
        return carry

    lax.fori_loop(0, CPT, chunk, 0)
    plsc.subcore_barrier()
    pltpu.sync_copy(acc_sh.at[pl.ds(base, RPT)], out_hbm.at[c, pl.ds(base, RPT)])


_sc_scatter = pl.kernel(
    _scat_body,
    mesh=_mesh,
    out_type=jax.ShapeDtypeStruct((2, NP, DH), jnp.float32),
    scratch_types=[
        pltpu.VMEM((CPT, CH), jnp.int32),
        pltpu.VMEM((CPT, CH), jnp.int32),
        pltpu.VMEM((CH,), jnp.int32),
        pltpu.VMEM((CH, DH), jnp.float32),
        pltpu.VMEM_SHARED((NP, DH), jnp.float32),
        pltpu.SemaphoreType.DMA,
    ],
)


def _mmT(a, w):
    return lax.dot_general(a, w, (((1,), (1,)), ((), ())),
                           preferred_element_type=jnp.float32)


def _pre_body(dacc_ref, x_ref, dinv_ref, ystk_ref):
    deg = dacc_ref[0, :, 0:1] + dacc_ref[1, :, 0:1] + 1.0
    dinv = lax.rsqrt(deg)
    dinv_ref[...] = dinv
    y = x_ref[...] * dinv
    ystk_ref[0] = y[:, :DH]
    ystk_ref[1] = y[:, DH:]


def _tc_pre(dacc, x):
    return pl.pallas_call(
        _pre_body,
        grid=(N // BM,),
        in_specs=[
            pl.BlockSpec((2, BM, DH), lambda i: (0, i, 0)),
            pl.BlockSpec((BM, D), lambda i: (i, 0)),
        ],
        out_specs=[
            pl.BlockSpec((BM, 1), lambda i: (i, 0)),
            pl.BlockSpec((2, BM, DH), lambda i: (0, i, 0)),
        ],
        out_shape=[
            jax.ShapeDtypeStruct((N, 1), jnp.float32),
            jax.ShapeDtypeStruct((2, N, DH), jnp.float32),
        ],
    )(dacc, x)


def _mid_body(acc_ref, ystk_ref, dinv_ref, w_ref, b_ref, out_ref):
    dinv = dinv_ref[...]
    g0 = (acc_ref[0] + ystk_ref[0]) * dinv
    g1 = (acc_ref[1] + ystk_ref[1]) * dinv
    w = w_ref[...]
    h = _mmT(g0, w[:, :DH]) + _mmT(g1, w[:, DH:]) + b_ref[...]
    y = jnp.maximum(h, 0.0) * dinv
    out_ref[0] = y[:, :DH]
    out_ref[1] = y[:, DH:]


def _tc_mid(acc, ystk, dinv, w, b):
    return pl.pallas_call(
        _mid_body,
        grid=(N // BM,),
        in_specs=[
            pl.BlockSpec((2, BM, DH), lambda i: (0, i, 0)),
            pl.BlockSpec((2, BM, DH), lambda i: (0, i, 0)),
            pl.BlockSpec((BM, 1), lambda i: (i, 0)),
            pl.BlockSpec((D, D), lambda i: (0, 0)),
            pl.BlockSpec((1, D), lambda i: (0, 0)),
        ],
        out_specs=pl.BlockSpec((2, BM, DH), lambda i: (0, i, 0)),
        out_shape=jax.ShapeDtypeStruct((2, N, DH), jnp.float32),
    )(acc, ystk, dinv, w, b)


def _fin_body(acc_ref, ystk_ref, dinv_ref, w_ref, b_ref, out_ref):
    dinv = dinv_ref[...]
    g0 = (acc_ref[0] + ystk_ref[0]) * dinv
    g1 = (acc_ref[1] + ystk_ref[1]) * dinv
    w = w_ref[...]
    out_ref[...] = _mmT(g0, w[:, :DH]) + _mmT(g1, w[:, DH:]) + b_ref[...]


def _tc_fin(acc, ystk, dinv, w, b):
    return pl.pallas_call(
        _fin_body,
        grid=(N // BM,),
        in_specs=[
            pl.BlockSpec((2, BM, DH), lambda i: (0, i, 0)),
            pl.BlockSpec((2, BM, DH), lambda i: (0, i, 0)),
            pl.BlockSpec((BM, 1), lambda i: (i, 0)),
            pl.BlockSpec((D, D), lambda i: (0, 0)),
            pl.BlockSpec((1, D), lambda i: (0, 0)),
        ],
        out_specs=pl.BlockSpec((BM, D), lambda i: (i, 0)),
        out_shape=jax.ShapeDtypeStruct((N, D), jnp.float32),
    )(acc, ystk, dinv, w, b)


def kernel(x, edge_index, edge_type, W1, b1, W2, b2, W3, b3):
    del edge_type
    row = edge_index[0].astype(jnp.int32)
    col = edge_index[1].astype(jnp.int32)
    pad = EP - E
    row_p = jnp.concatenate([row, jnp.zeros((pad,), jnp.int32)]).reshape(NCHUNK, CH)
    col_p = jnp.concatenate([col, jnp.full((pad,), N, jnp.int32)]).reshape(NCHUNK, CH)
    ones128 = jnp.ones((CH, DH), jnp.float32)
    zeros128 = jnp.zeros((NP, DH), jnp.float32)

    dacc = _sc_degree(col_p, ones128, zeros128)
    dinv, ystk = _tc_pre(dacc, x)

    for w, b, last in ((W1, b1, False), (W2, b2, False), (W3, b3, True)):
        table = ystk.reshape(2 * N, DH)
        acc = _sc_scatter(table, row_p, col_p, zeros128)
        if last:
            return _tc_fin(acc, ystk, dinv, w, b.reshape(1, D))
        ystk = _tc_mid(acc, ystk, dinv, w, b.reshape(1, D))
